# Initial kernel scaffold; baseline (speedup 1.0000x reference)
import jax
import jax.numpy as jnp
from jax.experimental import pallas as pl


def kernel(x, edge_index, W1, b1, gamma1, beta1, W2, b2, gamma2, beta2):
    raise NotImplementedError("write your pallas kernel here")


# speedup vs baseline: 1.0171x; 1.0171x over previous
import jax
import jax.numpy as jnp
from jax.experimental import pallas as pl
from jax.experimental.pallas import tpu as pltpu

N = 10000
K = 10
ALPHA = 0.1
EPS = 1e-5


def _dense_body(x_ref, W1_ref, b1_ref, g1_ref, be1_ref, W2_ref, b2_ref,
                g2_ref, be2_ref, out_ref):
    x = x_ref[...]
    h = jax.lax.dot_general(x, W1_ref[...], (((1,), (1,)), ((), ())),
                            preferred_element_type=jnp.float32)
    h = jnp.maximum(h + b1_ref[...][None, :], 0.0)
    mu = jnp.mean(h, axis=0, keepdims=True)
    var = jnp.mean((h - mu) ** 2, axis=0, keepdims=True)
    h = (h - mu) * jax.lax.rsqrt(var + EPS) * g1_ref[...][None, :] + be1_ref[...][None, :]
    h = jax.lax.dot_general(h, W2_ref[...], (((1,), (1,)), ((), ())),
                            preferred_element_type=jnp.float32)
    h = h + b2_ref[...][None, :]
    mu = jnp.mean(h, axis=0, keepdims=True)
    var = jnp.mean((h - mu) ** 2, axis=0, keepdims=True)
    out_ref[...] = (h - mu) * jax.lax.rsqrt(var + EPS) * g2_ref[...][None, :] + be2_ref[...][None, :]


def _mlp(x, W1, b1, gamma1, beta1, W2, b2, gamma2, beta2):
    return pl.pallas_call(
        _dense_body,
        out_shape=jax.ShapeDtypeStruct((N, 128), jnp.float32),
    )(x, W1, b1, gamma1, beta1, W2, b2, gamma2, beta2)


def kernel(x, edge_index, W1, b1, gamma1, beta1, W2, b2, gamma2, beta2):
    h = _mlp(x, W1, b1, gamma1, beta1, W2, b2, gamma2, beta2)
    n = x.shape[0]
    loop = jnp.arange(n, dtype=edge_index.dtype)
    src = jnp.concatenate([edge_index[0], loop])
    dst = jnp.concatenate([edge_index[1], loop])
    deg = jax.ops.segment_sum(jnp.ones(src.shape, dtype=h.dtype), dst, num_segments=n)
    dinv = jnp.where(deg > 0, jax.lax.rsqrt(deg), 0.0)
    norm = dinv[src] * dinv[dst]
    h0 = h
    for _ in range(K):
        msg = h[src] * norm[:, None]
        agg = jax.ops.segment_sum(msg, dst, num_segments=n)
        h = (1.0 - ALPHA) * agg + ALPHA * h0
    return h
